# baseline (speedup 1.0000x reference)
import jax
import jax.numpy as jnp
from jax import lax
from jax.experimental import pallas as pl
from jax.experimental.pallas import tpu as pltpu
from jax.experimental.pallas import tpu_sc as plsc

N = 1000
B = 16384
NC = 2
NS = 16
NW = NC * NS
BPW = B // NW
CHUNK = 128
NCH = BPW // CHUNK
NV = BPW // 16
RPAD = 1024
RB = 40
GSTEPS = N // RB
LBINS = GSTEPS * 128


def _tc_body(inp_ref, cab_ref, cba_ref,
             a_ref, b_ref, lA_ref, lB_ref, fA_ref, fB_ref):
    i = pl.program_id(0)

    @pl.when(i == 0)
    def _extract_ab():
        a_ref[:] = inp_ref[:, 0]
        b_ref[:] = inp_ref[:, 1]

    zpad = jnp.zeros((128 - RB,), jnp.float32)

    def do_table(c_ref, l_ref, f_ref):
        blk = c_ref[:]
        for r in range(RB):
            f_ref[pl.ds(r * RPAD, N)] = blk[r, :]
        m = jnp.max(blk, axis=1)
        lse = jnp.log(jnp.sum(jnp.exp(blk - m[:, None]), axis=1)) + m
        l_ref[:] = jnp.concatenate([lse, zpad])

    do_table(cab_ref, lA_ref, fA_ref)
    do_table(cba_ref, lB_ref, fB_ref)


_tc_call = pl.pallas_call(
    _tc_body,
    grid=(GSTEPS,),
    in_specs=[
        pl.BlockSpec((B, 2), lambda i: (0, 0)),
        pl.BlockSpec((RB, N), lambda i: (i, 0)),
        pl.BlockSpec((RB, N), lambda i: (i, 0)),
    ],
    out_specs=[
        pl.BlockSpec((B,), lambda i: (0,)),
        pl.BlockSpec((B,), lambda i: (0,)),
        pl.BlockSpec((128,), lambda i: (i,)),
        pl.BlockSpec((128,), lambda i: (i,)),
        pl.BlockSpec((RB * RPAD,), lambda i: (i,)),
        pl.BlockSpec((RB * RPAD,), lambda i: (i,)),
    ],
    out_shape=(
        jax.ShapeDtypeStruct((B,), jnp.int32),
        jax.ShapeDtypeStruct((B,), jnp.int32),
        jax.ShapeDtypeStruct((LBINS,), jnp.float32),
        jax.ShapeDtypeStruct((LBINS,), jnp.float32),
        jax.ShapeDtypeStruct((N * RPAD,), jnp.float32),
        jax.ShapeDtypeStruct((N * RPAD,), jnp.float32),
    ),
)


def _sc_body(a_hbm, b_hbm, wmA_hbm, wmB_hbm, fA_hbm, fB_hbm,
             outA_hbm, outB_hbm, cntA_hbm, cntB_hbm,
             a_v, b_v, idxA, idxB, idxTA, idxTB, idxHA, idxHB, gA, gB, gmA, gmB,
             ones_v, zeros_v, accA_v, accB_v, hist_a, hist_b, sem):
    cid = lax.axis_index("c")
    sid = lax.axis_index("s")
    wid = sid * NC + cid
    base = wid * BPW
    pltpu.sync_copy(a_hbm.at[pl.ds(base, BPW)], a_v)
    pltpu.sync_copy(b_hbm.at[pl.ds(base, BPW)], b_v)
    for k in range(8):
        ones_v[pl.ds(16 * k, 16)] = jnp.ones((16,), jnp.float32)

    @pl.when(sid == 0)
    def _zero_hist():
        for k in range(LBINS // 16):
            zeros_v[pl.ds(16 * k, 16)] = jnp.zeros((16,), jnp.float32)
        pltpu.sync_copy(zeros_v, hist_a)
        pltpu.sync_copy(zeros_v, hist_b)

    pair_cp = []
    for j in range(NV):
        a16 = a_v[pl.ds(16 * j, 16)]
        b16 = b_v[pl.ds(16 * j, 16)]
        r, s = j // 8, pl.ds(16 * (j % 8), 16)
        idxA[r, s] = (a16 << 10) + b16
        idxB[r, s] = (b16 << 10) + a16
        idxTA[r, s] = a16
        idxTB[r, s] = b16
    for c in range(NCH):
        pair_cp.append(pltpu.async_copy(fA_hbm.at[idxA.at[c]], gA.at[c], sem))
        pair_cp.append(pltpu.async_copy(fB_hbm.at[idxB.at[c]], gB.at[c], sem))
        pair_cp.append(pltpu.async_copy(wmA_hbm.at[idxTA.at[c]], gmA.at[c], sem))
        pair_cp.append(pltpu.async_copy(wmB_hbm.at[idxTB.at[c]], gmB.at[c], sem))

    for j in range(NV):
        r, s = j // 8, pl.ds(16 * (j % 8), 16)
        a16 = idxTA[r, s]
        b16 = idxTB[r, s]
        qa = (a16 * 52429) >> 21
        qb = (b16 * 52429) >> 21
        idxHA[r, s] = (qa << 7) + a16 - RB * qa
        idxHB[r, s] = (qb << 7) + b16 - RB * qb

    plsc.subcore_barrier()
    for c in range(NCH):
        pltpu.sync_copy(ones_v, hist_a.at[idxHA.at[c]], add=True)
        pltpu.sync_copy(ones_v, hist_b.at[idxHB.at[c]], add=True)
    plsc.subcore_barrier()

    @pl.when(sid == 0)
    def _write_hist():
        pltpu.sync_copy(hist_a, cntA_hbm.at[cid])
        pltpu.sync_copy(hist_b, cntB_hbm.at[cid])

    for cp in pair_cp:
        cp.wait()
    accA = jnp.zeros((16,), jnp.float32)
    accB = jnp.zeros((16,), jnp.float32)
    for j in range(NV):
        r, s = j // 8, pl.ds(16 * (j % 8), 16)
        accA = accA + gA[r, s] + gmA[r, s]
        accB = accB + gB[r, s] + gmB[r, s]
    accA_v[:] = accA
    accB_v[:] = accB
    pltpu.sync_copy(accA_v, outA_hbm.at[wid])
    pltpu.sync_copy(accB_v, outB_hbm.at[wid])


_sc_call = pl.kernel(
    _sc_body,
    out_type=(
        jax.ShapeDtypeStruct((NW, 16), jnp.float32),
        jax.ShapeDtypeStruct((NW, 16), jnp.float32),
        jax.ShapeDtypeStruct((NC, LBINS), jnp.float32),
        jax.ShapeDtypeStruct((NC, LBINS), jnp.float32),
    ),
    mesh=plsc.VectorSubcoreMesh(core_axis_name="c", subcore_axis_name="s"),
    scratch_types=(
        pltpu.VMEM((BPW,), jnp.int32),
        pltpu.VMEM((BPW,), jnp.int32),
        pltpu.VMEM((NCH, CHUNK), jnp.int32),
        pltpu.VMEM((NCH, CHUNK), jnp.int32),
        pltpu.VMEM((NCH, CHUNK), jnp.int32),
        pltpu.VMEM((NCH, CHUNK), jnp.int32),
        pltpu.VMEM((NCH, CHUNK), jnp.int32),
        pltpu.VMEM((NCH, CHUNK), jnp.int32),
        pltpu.VMEM((NCH, CHUNK), jnp.float32),
        pltpu.VMEM((NCH, CHUNK), jnp.float32),
        pltpu.VMEM((NCH, CHUNK), jnp.float32),
        pltpu.VMEM((NCH, CHUNK), jnp.float32),
        pltpu.VMEM((CHUNK,), jnp.float32),
        pltpu.VMEM((LBINS,), jnp.float32),
        pltpu.VMEM((16,), jnp.float32),
        pltpu.VMEM((16,), jnp.float32),
        pltpu.VMEM_SHARED((LBINS,), jnp.float32),
        pltpu.VMEM_SHARED((LBINS,), jnp.float32),
        pltpu.SemaphoreType.DMA,
    ),
)


def _combine_body(w_ref, wmA_ref, wmB_ref, lA_ref, lB_ref,
                  cntA_ref, cntB_ref, pA_ref, pB_ref, out_ref):
    def lse1d(v):
        m = jnp.max(v)
        return jnp.log(jnp.sum(jnp.exp(v - m))) + m

    cA = cntA_ref[0, :] + cntA_ref[1, :]
    cB = cntB_ref[0, :] + cntB_ref[1, :]
    S_AB = jnp.sum(pA_ref[:]) - B * lse1d(wmA_ref[:]) - jnp.sum(cA * lA_ref[:])
    S_BA = jnp.sum(pB_ref[:]) - B * lse1d(wmB_ref[:]) - jnp.sum(cB * lB_ref[:])
    wv = w_ref[:, :]
    la = -jnp.log(1.0 + jnp.exp(-wv))
    l1a = -jnp.log(1.0 + jnp.exp(wv))
    x = la + S_AB
    y = l1a + S_BA
    m = jnp.maximum(x, y)
    out_ref[:, :] = m + jnp.log(jnp.exp(x - m) + jnp.exp(y - m))


_combine_call = pl.pallas_call(
    _combine_body,
    out_shape=jax.ShapeDtypeStruct((1, 1), jnp.float32),
)


def kernel(inputs, w, w_mA, w_cAB, w_mB, w_cBA):
    a, b, lA, lB, fA, fB = _tc_call(inputs, w_cAB, w_cBA)
    outA, outB, cntA, cntB = _sc_call(a, b, w_mA, w_mB, fA, fB)
    res = _combine_call(jnp.reshape(w, (1, 1)), w_mA, w_mB, lA, lB,
                        cntA, cntB, outA, outB)
    return jnp.reshape(res, ())
